# baseline (speedup 1.0000x reference)
import functools

import jax
import jax.numpy as jnp
from jax import lax
from jax.experimental import pallas as pl
from jax.experimental.pallas import tpu as pltpu
from jax.experimental.pallas import tpu_sc as plsc

B = 4096
L = 50
D = 64
DP = 128
BL = B * L
VOCAB = 1000000

_info = plsc.get_sparse_core_info()
NC, NS, LANES = _info.num_cores, _info.num_subcores, _info.num_lanes
NW = NC * NS
ROWS_PER_W = B // NW
CHUNK = 100
CHUNKP = 112
NCHUNK = ROWS_PER_W // 2


def _make_kernel():
    mesh = plsc.VectorSubcoreMesh(core_axis_name="c", subcore_axis_name="s")

    @functools.partial(
        pl.kernel,
        mesh=mesh,
        out_type=jax.ShapeDtypeStruct((B, L, D), jnp.float32),
        compiler_params=pltpu.CompilerParams(use_tc_tiling_on_sc=False),
        scratch_types=[
            pltpu.VMEM((NCHUNK, CHUNKP), jnp.int32),
            pltpu.VMEM((NCHUNK, CHUNKP), jnp.float32),
            pltpu.VMEM((CHUNKP, DP), jnp.float32),
            pltpu.VMEM((CHUNKP, D), jnp.float32),
            pltpu.SemaphoreType.DMA,
            pltpu.SemaphoreType.DMA,
        ],
    )
    def k(tok_hbm, mask_hbm, table_hbm, out_hbm,
          tok_v, mask_v, rows_v, cmp_v, gsem, osem):
        w = lax.axis_index("s") * NC + lax.axis_index("c")
        pltpu.sync_copy(tok_hbm.at[w], tok_v)
        pltpu.sync_copy(mask_hbm.at[w], mask_v)

        def chunk_body(c, carry):
            pltpu.async_copy(table_hbm.at[tok_v.at[c]], rows_v, gsem).wait()

            for g in range(CHUNKP // LANES):
                mk16 = mask_v[c, pl.ds(g * LANES, LANES)]
                for j in range(LANES):
                    t = g * LANES + j
                    m = lax.gather(
                        mk16,
                        jnp.full((LANES, 1), j, jnp.int32),
                        lax.GatherDimensionNumbers(
                            offset_dims=(), collapsed_slice_dims=(0,),
                            start_index_map=(0,)),
                        (1,),
                        mode=lax.GatherScatterMode.PROMISE_IN_BOUNDS)
                    for q in range(D // LANES):
                        sl = pl.ds(q * LANES, LANES)
                        cmp_v[t, sl] = rows_v[t, sl] * m

            b0 = w * ROWS_PER_W + 2 * c
            pltpu.async_copy(cmp_v.at[pl.ds(0, L)], out_hbm.at[b0], osem).wait()
            pltpu.async_copy(cmp_v.at[pl.ds(L, L)], out_hbm.at[b0 + 1],
                             osem).wait()
            return carry

        lax.fori_loop(0, NCHUNK, chunk_body, 0)

    return k


_sc_kernel = _make_kernel()


def kernel(token_ids, seq_lens, glove_table):
    table_p = jnp.pad(glove_table, ((0, 0), (0, DP - D)))
    tok = token_ids.reshape(NW, NCHUNK, CHUNK).astype(jnp.int32)
    tok_p = jnp.pad(tok, ((0, 0), (0, 0), (0, CHUNKP - CHUNK)))
    mask = (jnp.arange(L, dtype=jnp.int32)[None, :]
            < seq_lens.astype(jnp.int32)[:, None]).astype(jnp.float32)
    mask_p = jnp.pad(mask.reshape(NW, NCHUNK, CHUNK),
                     ((0, 0), (0, 0), (0, CHUNKP - CHUNK)))
    return _sc_kernel(tok_p, mask_p, table_p)


# speedup vs baseline: 2.3684x; 2.3684x over previous
import functools

import jax
import jax.numpy as jnp
from jax import lax
from jax.experimental import pallas as pl
from jax.experimental.pallas import tpu as pltpu
from jax.experimental.pallas import tpu_sc as plsc

B = 4096
L = 50
D = 64
DP = 128
BL = B * L
VOCAB = 1000000

_info = plsc.get_sparse_core_info()
NC, NS, LANES = _info.num_cores, _info.num_subcores, _info.num_lanes
NW = NC * NS
TOK_PER_W = BL // NW
CHUNK = 128
NCHUNK = TOK_PER_W // CHUNK
NPAIR = NCHUNK // 2


def _make_kernel():
    mesh = plsc.VectorSubcoreMesh(core_axis_name="c", subcore_axis_name="s")

    @functools.partial(
        pl.kernel,
        mesh=mesh,
        out_type=jax.ShapeDtypeStruct((BL, D), jnp.float32),
        compiler_params=pltpu.CompilerParams(use_tc_tiling_on_sc=False),
        scratch_types=[
            pltpu.VMEM((NCHUNK, CHUNK), jnp.int32),
            pltpu.VMEM((NCHUNK, CHUNK), jnp.float32),
            pltpu.VMEM((CHUNK, DP), jnp.float32),
            pltpu.VMEM((CHUNK, DP), jnp.float32),
            pltpu.VMEM((CHUNK, D), jnp.float32),
            pltpu.VMEM((CHUNK, D), jnp.float32),
            pltpu.SemaphoreType.DMA,
            pltpu.SemaphoreType.DMA,
            pltpu.SemaphoreType.DMA,
            pltpu.SemaphoreType.DMA,
        ],
    )
    def k(tok_hbm, mask_hbm, table_hbm, out_hbm,
          tok_v, mask_v, rows_a, rows_b, cmp_a, cmp_b,
          gsem_a, gsem_b, osem_a, osem_b):
        w = lax.axis_index("s") * NC + lax.axis_index("c")
        base_w = w * TOK_PER_W
        pltpu.sync_copy(tok_hbm.at[w], tok_v)
        pltpu.sync_copy(mask_hbm.at[w], mask_v)

        def gstart(c, rows, gsem):
            pltpu.async_copy(table_hbm.at[tok_v.at[c]], rows, gsem)

        def gwait(c, rows, gsem):
            pltpu.make_async_copy(table_hbm.at[tok_v.at[c]], rows, gsem).wait()

        def ostart(c, cmp, osem):
            pltpu.async_copy(cmp, out_hbm.at[pl.ds(base_w + c * CHUNK, CHUNK)],
                             osem)

        def owait(cmp, osem):
            pltpu.make_async_copy(cmp, out_hbm.at[pl.ds(base_w, CHUNK)],
                                  osem).wait()

        def compute(c, rows, cmp):
            for g in range(CHUNK // LANES):
                mk16 = mask_v[c, pl.ds(g * LANES, LANES)]
                for j in range(LANES):
                    t = g * LANES + j
                    m = lax.gather(
                        mk16,
                        jnp.full((LANES, 1), j, jnp.int32),
                        lax.GatherDimensionNumbers(
                            offset_dims=(), collapsed_slice_dims=(0,),
                            start_index_map=(0,)),
                        (1,),
                        mode=lax.GatherScatterMode.PROMISE_IN_BOUNDS)
                    for q in range(D // LANES):
                        sl = pl.ds(q * LANES, LANES)
                        cmp[t, sl] = rows[t, sl] * m

        gstart(0, rows_a, gsem_a)
        gstart(1, rows_b, gsem_b)

        def half(p, c, rows, cmp, gsem, osem):
            gwait(c, rows, gsem)

            @pl.when(p > 0)
            def _():
                owait(cmp, osem)

            compute(c, rows, cmp)

            @pl.when(c + 2 < NCHUNK)
            def _():
                gstart(c + 2, rows, gsem)

            ostart(c, cmp, osem)

        def pair_body(p, carry):
            half(p, 2 * p, rows_a, cmp_a, gsem_a, osem_a)
            half(p, 2 * p + 1, rows_b, cmp_b, gsem_b, osem_b)
            return carry

        lax.fori_loop(0, NPAIR, pair_body, 0)
        owait(cmp_a, osem_a)
        owait(cmp_b, osem_b)

    return k


_sc_kernel = _make_kernel()


def kernel(token_ids, seq_lens, glove_table):
    table_p = jnp.pad(glove_table, ((0, 0), (0, DP - D)))
    tok = token_ids.reshape(NW, NCHUNK, CHUNK).astype(jnp.int32)
    mask = (jnp.arange(L, dtype=jnp.int32)[None, :]
            < seq_lens.astype(jnp.int32)[:, None]).astype(jnp.float32)
    mask3d = mask.reshape(NW, NCHUNK, CHUNK)
    out = _sc_kernel(tok, mask3d, table_p)
    return out.reshape(B, L, D)
